# baseline (speedup 1.0000x reference)
import functools

import jax
import jax.numpy as jnp
from jax import lax
from jax.experimental import pallas as pl
from jax.experimental.pallas import tpu as pltpu
from jax.experimental.pallas import tpu_sc as plsc

B = 1024
SIZE = 26
D = 4096
NLEV = 100
LPAD = 104
TROWS = SIZE * LPAD

NC = 2
NS = 16
NW = NC * NS
B_PER_W = B // NW

SL = 32
LN = D // SL
SLW = 16
LNW = 128
LANES = 16


def _table_body(pos_ref, lev_ref, t_ref):
    p = pos_ref[0]
    lev = lev_ref[...]
    t = p + lev * (1.0 - 2.0 * p)
    t_ref[...] = t.astype(jnp.int16).reshape(LPAD, SL, LN)


def _build_table(position, levels_pad):
    pos3 = position.reshape(SIZE, 1, D)
    return pl.pallas_call(
        _table_body,
        grid=(SIZE,),
        in_specs=[
            pl.BlockSpec((1, 1, D), lambda s: (s, 0, 0)),
            pl.BlockSpec((LPAD, D), lambda s: (0, 0)),
        ],
        out_specs=pl.BlockSpec((LPAD, SL, LN), lambda s: (s, 0, 0)),
        out_shape=jax.ShapeDtypeStruct((TROWS, SL, LN), jnp.int16),
    )(pos3, levels_pad)


def _fidx_body(x_ref, out_ref):
    xv = x_ref[...]
    q = jnp.floor(xv * float(NLEV))
    q = jnp.clip(q, 0.0, float(NLEV - 1)).astype(jnp.int32)
    s = lax.broadcasted_iota(jnp.int32, (B, SIZE), 1)
    out_ref[...] = q + s * LPAD


def _build_fidx(x):
    return pl.pallas_call(
        _fidx_body,
        in_specs=[pl.BlockSpec((B, SIZE), lambda: (0, 0))],
        out_specs=pl.BlockSpec((B, SIZE), lambda: (0, 0)),
        out_shape=jax.ShapeDtypeStruct((B, SIZE), jnp.int32),
    )(x)


def _sc_gather_sum(table3, fidx):
    mesh = plsc.VectorSubcoreMesh(core_axis_name="c", subcore_axis_name="s")

    @functools.partial(
        pl.kernel,
        mesh=mesh,
        out_type=jax.ShapeDtypeStruct((B, SLW, LNW), jnp.int32),
        scratch_types=[
            pltpu.VMEM((B_PER_W, SIZE), jnp.int32),
            pltpu.VMEM((SIZE, SLW, LNW), jnp.int32),
            pltpu.VMEM((SIZE, SLW, LNW), jnp.int32),
            pltpu.VMEM((SLW, LNW), jnp.int32),
            pltpu.SemaphoreType.DMA,
            pltpu.SemaphoreType.DMA,
        ],
    )
    def k(table_hbm, fidx_hbm, out_hbm, idx_v, rows_a, rows_b, outrow_v,
          sem_a, sem_b):
        wid = lax.axis_index("s") * NC + lax.axis_index("c")
        base = wid * B_PER_W
        pltpu.sync_copy(fidx_hbm.at[pl.ds(base, B_PER_W)], idx_v)

        def accumulate(rows_v, j):
            def chunk_body(c, carry2):
                off = c * LANES
                for sl in range(SLW):
                    acc = rows_v[0, sl, pl.ds(off, LANES)]
                    for s in range(1, SIZE):
                        acc = acc + rows_v[s, sl, pl.ds(off, LANES)]
                    outrow_v[sl, pl.ds(off, LANES)] = acc
                return carry2

            lax.fori_loop(0, LNW // LANES, chunk_body, 0, unroll=False)
            pltpu.sync_copy(outrow_v, out_hbm.at[base + j])

        pltpu.async_copy(table_hbm.at[idx_v.at[0]], rows_a, sem_a)

        def step(i, carry):
            j = 2 * i
            nxt_b = jnp.minimum(j + 1, B_PER_W - 1)
            cp_b = pltpu.async_copy(table_hbm.at[idx_v.at[nxt_b]], rows_b, sem_b)
            pltpu.make_async_copy(table_hbm.at[idx_v.at[j]], rows_a, sem_a).wait()
            accumulate(rows_a, j)
            nxt_a = jnp.minimum(j + 2, B_PER_W - 1)
            cp_a = pltpu.async_copy(table_hbm.at[idx_v.at[nxt_a]], rows_a, sem_a)
            cp_b.wait()
            accumulate(rows_b, j + 1)
            return carry

        lax.fori_loop(0, B_PER_W // 2, step, 0, unroll=False)
        pltpu.make_async_copy(table_hbm.at[idx_v.at[B_PER_W - 1]], rows_a,
                              sem_a).wait()

    return k(table3, fidx)


def kernel(x, position, levels):
    levels_pad = jnp.pad(levels, ((0, LPAD - NLEV), (0, 0)))
    table = _build_table(position, levels_pad)
    table_w = lax.bitcast_convert_type(
        table.reshape(TROWS, SLW, LNW, 2), jnp.int32)
    fidx = _build_fidx(x)
    acc_w = _sc_gather_sum(table_w, fidx)
    acc16 = lax.bitcast_convert_type(acc_w, jnp.int16)
    return acc16.reshape(B, D).astype(jnp.float32)


# speedup vs baseline: 2.4198x; 2.4198x over previous
import functools

import jax
import jax.numpy as jnp
from jax import lax
from jax.experimental import pallas as pl
from jax.experimental.pallas import tpu as pltpu
from jax.experimental.pallas import tpu_sc as plsc

B = 1024
SIZE = 26
D = 4096
NLEV = 100
LPAD = 104
TROWS = SIZE * LPAD
DH = D // 2

NC = 2
NS = 16
NW = NC * NS
B_PER_W = B // NW

SLW = 16
LNW = 128
LANES = 16


def _table_body(pos_lo_ref, pos_hi_ref, lev_lo_ref, lev_hi_ref, t_ref):
    p_lo = pos_lo_ref[0]
    p_hi = pos_hi_ref[0]
    l_lo = lev_lo_ref[...]
    l_hi = lev_hi_ref[...]
    t_lo = p_lo + l_lo * (1.0 - 2.0 * p_lo)
    t_hi = p_hi + l_hi * (1.0 - 2.0 * p_hi)
    t_ref[...] = t_lo.astype(jnp.int32) + t_hi.astype(jnp.int32) * 65536


def _build_table(pos_lo, pos_hi, lev_lo, lev_hi):
    return pl.pallas_call(
        _table_body,
        grid=(SIZE,),
        in_specs=[
            pl.BlockSpec((1, 1, SLW, LNW), lambda s: (s, 0, 0, 0)),
            pl.BlockSpec((1, 1, SLW, LNW), lambda s: (s, 0, 0, 0)),
            pl.BlockSpec((LPAD, SLW, LNW), lambda s: (0, 0, 0)),
            pl.BlockSpec((LPAD, SLW, LNW), lambda s: (0, 0, 0)),
        ],
        out_specs=pl.BlockSpec((LPAD, SLW, LNW), lambda s: (s, 0, 0)),
        out_shape=jax.ShapeDtypeStruct((TROWS, SLW, LNW), jnp.int32),
    )(pos_lo, pos_hi, lev_lo, lev_hi)


def _fidx_body(x_ref, out_ref):
    xv = x_ref[...]
    q = jnp.floor(xv * float(NLEV))
    q = jnp.clip(q, 0.0, float(NLEV - 1)).astype(jnp.int32)
    s = lax.broadcasted_iota(jnp.int32, (B, SIZE), 1)
    out_ref[...] = q + s * LPAD


def _build_fidx(x):
    return pl.pallas_call(
        _fidx_body,
        in_specs=[pl.BlockSpec((B, SIZE), lambda: (0, 0))],
        out_specs=pl.BlockSpec((B, SIZE), lambda: (0, 0)),
        out_shape=jax.ShapeDtypeStruct((B, SIZE), jnp.int32),
    )(x)


def _unpack_body(w_ref, out_ref):
    w = w_ref[...]
    out_ref[:, 0] = (w & 0xFFFF).astype(jnp.float32)
    out_ref[:, 1] = (w >> 16).astype(jnp.float32)


def _unpack(acc_w):
    blk = 256
    out4 = pl.pallas_call(
        _unpack_body,
        grid=(B // blk,),
        in_specs=[pl.BlockSpec((blk, SLW, LNW), lambda i: (i, 0, 0))],
        out_specs=pl.BlockSpec((blk, 2, SLW, LNW), lambda i: (i, 0, 0, 0)),
        out_shape=jax.ShapeDtypeStruct((B, 2, SLW, LNW), jnp.float32),
    )(acc_w)
    return out4.reshape(B, D)


def _sc_gather_sum(table_w, fidx):
    mesh = plsc.VectorSubcoreMesh(core_axis_name="c", subcore_axis_name="s")

    @functools.partial(
        pl.kernel,
        mesh=mesh,
        out_type=jax.ShapeDtypeStruct((B, SLW, LNW), jnp.int32),
        scratch_types=[
            pltpu.VMEM((B_PER_W, SIZE), jnp.int32),
            pltpu.VMEM((SIZE, SLW, LNW), jnp.int32),
            pltpu.VMEM((SIZE, SLW, LNW), jnp.int32),
            pltpu.VMEM((SLW, LNW), jnp.int32),
            pltpu.SemaphoreType.DMA,
            pltpu.SemaphoreType.DMA,
        ],
    )
    def k(table_hbm, fidx_hbm, out_hbm, idx_v, rows_a, rows_b, outrow_v,
          sem_a, sem_b):
        wid = lax.axis_index("s") * NC + lax.axis_index("c")
        base = wid * B_PER_W
        pltpu.sync_copy(fidx_hbm.at[pl.ds(base, B_PER_W)], idx_v)

        def accumulate(rows_v, j):
            def chunk_body(c, carry2):
                off = c * LANES
                for sl in range(SLW):
                    acc = rows_v[0, sl, pl.ds(off, LANES)]
                    for s in range(1, SIZE):
                        acc = acc + rows_v[s, sl, pl.ds(off, LANES)]
                    outrow_v[sl, pl.ds(off, LANES)] = acc
                return carry2

            lax.fori_loop(0, LNW // LANES, chunk_body, 0, unroll=False)
            pltpu.sync_copy(outrow_v, out_hbm.at[base + j])

        pltpu.async_copy(table_hbm.at[idx_v.at[0]], rows_a, sem_a)

        def step(i, carry):
            j = 2 * i
            nxt_b = jnp.minimum(j + 1, B_PER_W - 1)
            cp_b = pltpu.async_copy(table_hbm.at[idx_v.at[nxt_b]], rows_b,
                                    sem_b)
            pltpu.make_async_copy(table_hbm.at[idx_v.at[j]], rows_a,
                                  sem_a).wait()
            accumulate(rows_a, j)
            nxt_a = jnp.minimum(j + 2, B_PER_W - 1)
            pltpu.async_copy(table_hbm.at[idx_v.at[nxt_a]], rows_a, sem_a)
            cp_b.wait()
            accumulate(rows_b, j + 1)
            return carry

        lax.fori_loop(0, B_PER_W // 2, step, 0, unroll=False)
        pltpu.make_async_copy(table_hbm.at[idx_v.at[B_PER_W - 1]], rows_a,
                              sem_a).wait()

    return k(table_w, fidx)


def kernel(x, position, levels):
    levels_pad = jnp.pad(levels, ((0, LPAD - NLEV), (0, 0)))
    pos_lo = position[:, :DH].reshape(SIZE, 1, SLW, LNW)
    pos_hi = position[:, DH:].reshape(SIZE, 1, SLW, LNW)
    lev_lo = levels_pad[:, :DH].reshape(LPAD, SLW, LNW)
    lev_hi = levels_pad[:, DH:].reshape(LPAD, SLW, LNW)
    table_w = _build_table(pos_lo, pos_hi, lev_lo, lev_hi)
    fidx = _build_fidx(x)
    acc_w = _sc_gather_sum(table_w, fidx)
    return _unpack(acc_w)


# speedup vs baseline: 4.4516x; 1.8397x over previous
import functools

import jax
import jax.numpy as jnp
from jax import lax
from jax.experimental import pallas as pl
from jax.experimental.pallas import tpu as pltpu
from jax.experimental.pallas import tpu_sc as plsc

B = 1024
SIZE = 26
D = 4096
NLEV = 100
LPAD = 104
TROWS = SIZE * LPAD

NC = 2
NS = 16
NW = NC * NS
B_PER_W = B // NW

NQ = 4
SLW = 8
LNW = 128
LANES = 16


def _table_body(pos_ref, lev_ref, t_ref):
    p4 = pos_ref[0]
    l4 = lev_ref[...]
    w = jnp.zeros((LPAD, SLW, LNW), jnp.int32)
    for q in range(NQ):
        p = p4[q]
        l = l4[:, q]
        t = p + l * (1.0 - 2.0 * p)
        w = w + t.astype(jnp.int32) * (1 << (8 * q))
    t_ref[...] = w


def _build_table(pos4, lev4):
    return pl.pallas_call(
        _table_body,
        grid=(SIZE,),
        in_specs=[
            pl.BlockSpec((1, NQ, SLW, LNW), lambda s: (s, 0, 0, 0)),
            pl.BlockSpec((LPAD, NQ, SLW, LNW), lambda s: (0, 0, 0, 0)),
        ],
        out_specs=pl.BlockSpec((LPAD, SLW, LNW), lambda s: (s, 0, 0)),
        out_shape=jax.ShapeDtypeStruct((TROWS, SLW, LNW), jnp.int32),
    )(pos4, lev4)


def _fidx_body(x_ref, out_ref):
    xv = x_ref[...]
    q = jnp.floor(xv * float(NLEV))
    q = jnp.clip(q, 0.0, float(NLEV - 1)).astype(jnp.int32)
    s = lax.broadcasted_iota(jnp.int32, (B, SIZE), 1)
    out_ref[...] = q + s * LPAD


def _build_fidx(x):
    return pl.pallas_call(
        _fidx_body,
        in_specs=[pl.BlockSpec((B, SIZE), lambda: (0, 0))],
        out_specs=pl.BlockSpec((B, SIZE), lambda: (0, 0)),
        out_shape=jax.ShapeDtypeStruct((B, SIZE), jnp.int32),
    )(x)


def _unpack_body(w_ref, out_ref):
    w = w_ref[...]
    for q in range(NQ):
        out_ref[:, q] = ((w >> (8 * q)) & 0xFF).astype(jnp.float32)


def _unpack(acc_w):
    blk = 256
    out4 = pl.pallas_call(
        _unpack_body,
        grid=(B // blk,),
        in_specs=[pl.BlockSpec((blk, SLW, LNW), lambda i: (i, 0, 0))],
        out_specs=pl.BlockSpec((blk, NQ, SLW, LNW), lambda i: (i, 0, 0, 0)),
        out_shape=jax.ShapeDtypeStruct((B, NQ, SLW, LNW), jnp.float32),
    )(acc_w)
    return out4.reshape(B, D)


def _sum_tree(vals):
    while len(vals) > 1:
        nxt = [a + b for a, b in zip(vals[0::2], vals[1::2])]
        if len(vals) % 2:
            nxt.append(vals[-1])
        vals = nxt
    return vals[0]


def _sc_gather_sum(table_w, fidx):
    mesh = plsc.VectorSubcoreMesh(core_axis_name="c", subcore_axis_name="s")

    @functools.partial(
        pl.kernel,
        mesh=mesh,
        out_type=jax.ShapeDtypeStruct((B, SLW, LNW), jnp.int32),
        scratch_types=[
            pltpu.VMEM((B_PER_W, SIZE), jnp.int32),
            pltpu.VMEM((SIZE, SLW, LNW), jnp.int32),
            pltpu.VMEM((SIZE, SLW, LNW), jnp.int32),
            pltpu.VMEM((SLW, LNW), jnp.int32),
            pltpu.SemaphoreType.DMA,
            pltpu.SemaphoreType.DMA,
        ],
    )
    def k(table_hbm, fidx_hbm, out_hbm, idx_v, rows_a, rows_b, outrow_v,
          sem_a, sem_b):
        wid = lax.axis_index("s") * NC + lax.axis_index("c")
        base = wid * B_PER_W
        pltpu.sync_copy(fidx_hbm.at[pl.ds(base, B_PER_W)], idx_v)

        def accumulate(rows_v, j):
            def chunk_body(c, carry2):
                off = c * LANES
                for sl in range(SLW):
                    acc = _sum_tree(
                        [rows_v[s, sl, pl.ds(off, LANES)]
                         for s in range(SIZE)])
                    outrow_v[sl, pl.ds(off, LANES)] = acc
                return carry2

            lax.fori_loop(0, LNW // LANES, chunk_body, 0, unroll=False)
            pltpu.sync_copy(outrow_v, out_hbm.at[base + j])

        pltpu.async_copy(table_hbm.at[idx_v.at[0]], rows_a, sem_a)

        def step(i, carry):
            j = 2 * i
            nxt_b = jnp.minimum(j + 1, B_PER_W - 1)
            cp_b = pltpu.async_copy(table_hbm.at[idx_v.at[nxt_b]], rows_b,
                                    sem_b)
            pltpu.make_async_copy(table_hbm.at[idx_v.at[j]], rows_a,
                                  sem_a).wait()
            accumulate(rows_a, j)
            nxt_a = jnp.minimum(j + 2, B_PER_W - 1)
            pltpu.async_copy(table_hbm.at[idx_v.at[nxt_a]], rows_a, sem_a)
            cp_b.wait()
            accumulate(rows_b, j + 1)
            return carry

        lax.fori_loop(0, B_PER_W // 2, step, 0, unroll=False)
        pltpu.make_async_copy(table_hbm.at[idx_v.at[B_PER_W - 1]], rows_a,
                              sem_a).wait()

    return k(table_w, fidx)


def kernel(x, position, levels):
    levels_pad = jnp.pad(levels, ((0, LPAD - NLEV), (0, 0)))
    pos4 = position.reshape(SIZE, NQ, SLW, LNW)
    lev4 = levels_pad.reshape(LPAD, NQ, SLW, LNW)
    table_w = _build_table(pos4, lev4)
    fidx = _build_fidx(x)
    acc_w = _sc_gather_sum(table_w, fidx)
    return _unpack(acc_w)


# speedup vs baseline: 5.0060x; 1.1245x over previous
import functools

import jax
import jax.numpy as jnp
from jax import lax
from jax.experimental import pallas as pl
from jax.experimental.pallas import tpu as pltpu
from jax.experimental.pallas import tpu_sc as plsc

B = 1024
SIZE = 26
D = 4096
NLEV = 100
LPAD = 104
TROWS = SIZE * LPAD

NC = 2
NS = 16
NW = NC * NS
B_PER_W = B // NW

NQ = 4
SLW = 8
LNW = 128
LANES = 16


def _table_body(pos_ref, lev_ref, t_ref):
    p4 = pos_ref[0]
    l4 = lev_ref[...]
    w = jnp.zeros((LPAD, SLW, LNW), jnp.int32)
    for q in range(NQ):
        p = p4[q]
        l = l4[:, q]
        t = p + l * (1.0 - 2.0 * p)
        w = w + t.astype(jnp.int32) * (1 << (8 * q))
    t_ref[...] = w


def _build_table(pos4, lev4):
    return pl.pallas_call(
        _table_body,
        grid=(SIZE,),
        in_specs=[
            pl.BlockSpec((1, NQ, SLW, LNW), lambda s: (s, 0, 0, 0)),
            pl.BlockSpec((LPAD, NQ, SLW, LNW), lambda s: (0, 0, 0, 0)),
        ],
        out_specs=pl.BlockSpec((LPAD, SLW, LNW), lambda s: (s, 0, 0)),
        out_shape=jax.ShapeDtypeStruct((TROWS, SLW, LNW), jnp.int32),
    )(pos4, lev4)


def _fidx_body(x_ref, out_ref):
    xv = x_ref[...]
    q = jnp.floor(xv * float(NLEV))
    q = jnp.clip(q, 0.0, float(NLEV - 1)).astype(jnp.int32)
    s = lax.broadcasted_iota(jnp.int32, (B, SIZE), 1)
    out_ref[...] = q + s * LPAD


def _build_fidx(x):
    return pl.pallas_call(
        _fidx_body,
        in_specs=[pl.BlockSpec((B, SIZE), lambda: (0, 0))],
        out_specs=pl.BlockSpec((B, SIZE), lambda: (0, 0)),
        out_shape=jax.ShapeDtypeStruct((B, SIZE), jnp.int32),
    )(x)


def _unpack_body(w_ref, out_ref):
    w = w_ref[...]
    blk = w.shape[0]
    for q in range(NQ):
        f = ((w >> (8 * q)) & 0xFF).astype(jnp.float32)
        out_ref[:, q * (D // NQ):(q + 1) * (D // NQ)] = f.reshape(
            blk, D // NQ)


def _unpack(acc_w):
    blk = 256
    return pl.pallas_call(
        _unpack_body,
        grid=(B // blk,),
        in_specs=[pl.BlockSpec((blk, SLW, LNW), lambda i: (i, 0, 0))],
        out_specs=pl.BlockSpec((blk, D), lambda i: (i, 0)),
        out_shape=jax.ShapeDtypeStruct((B, D), jnp.float32),
    )(acc_w)


def _sum_tree(vals):
    while len(vals) > 1:
        nxt = [a + b for a, b in zip(vals[0::2], vals[1::2])]
        if len(vals) % 2:
            nxt.append(vals[-1])
        vals = nxt
    return vals[0]


def _sc_gather_sum(table_w, fidx):
    mesh = plsc.VectorSubcoreMesh(core_axis_name="c", subcore_axis_name="s")

    @functools.partial(
        pl.kernel,
        mesh=mesh,
        out_type=jax.ShapeDtypeStruct((B, SLW, LNW), jnp.int32),
        scratch_types=[
            pltpu.VMEM((B_PER_W, SIZE), jnp.int32),
            pltpu.VMEM((SIZE, SLW, LNW), jnp.int32),
            pltpu.VMEM((SIZE, SLW, LNW), jnp.int32),
            pltpu.VMEM((SLW, LNW), jnp.int32),
            pltpu.SemaphoreType.DMA,
            pltpu.SemaphoreType.DMA,
        ],
    )
    def k(table_hbm, fidx_hbm, out_hbm, idx_v, rows_a, rows_b, outrow_v,
          sem_a, sem_b):
        wid = lax.axis_index("s") * NC + lax.axis_index("c")
        base = wid * B_PER_W
        pltpu.sync_copy(fidx_hbm.at[pl.ds(base, B_PER_W)], idx_v)

        def accumulate(rows_v, j):
            def chunk_body(c, carry2):
                off = c * LANES
                for sl in range(SLW):
                    acc = _sum_tree(
                        [rows_v[s, sl, pl.ds(off, LANES)]
                         for s in range(SIZE)])
                    outrow_v[sl, pl.ds(off, LANES)] = acc
                return carry2

            lax.fori_loop(0, LNW // LANES, chunk_body, 0, unroll=False)
            pltpu.sync_copy(outrow_v, out_hbm.at[base + j])

        pltpu.async_copy(table_hbm.at[idx_v.at[0]], rows_a, sem_a)

        def step(i, carry):
            j = 2 * i
            nxt_b = jnp.minimum(j + 1, B_PER_W - 1)
            cp_b = pltpu.async_copy(table_hbm.at[idx_v.at[nxt_b]], rows_b,
                                    sem_b)
            pltpu.make_async_copy(table_hbm.at[idx_v.at[j]], rows_a,
                                  sem_a).wait()
            accumulate(rows_a, j)
            nxt_a = jnp.minimum(j + 2, B_PER_W - 1)
            pltpu.async_copy(table_hbm.at[idx_v.at[nxt_a]], rows_a, sem_a)
            cp_b.wait()
            accumulate(rows_b, j + 1)
            return carry

        lax.fori_loop(0, B_PER_W // 2, step, 0, unroll=False)
        pltpu.make_async_copy(table_hbm.at[idx_v.at[B_PER_W - 1]], rows_a,
                              sem_a).wait()

    return k(table_w, fidx)


def kernel(x, position, levels):
    levels_pad = jnp.pad(levels, ((0, LPAD - NLEV), (0, 0)))
    pos4 = position.reshape(SIZE, NQ, SLW, LNW)
    lev4 = levels_pad.reshape(LPAD, NQ, SLW, LNW)
    table_w = _build_table(pos4, lev4)
    fidx = _build_fidx(x)
    acc_w = _sc_gather_sum(table_w, fidx)
    return _unpack(acc_w)
